# Initial kernel scaffold; baseline (speedup 1.0000x reference)
import jax
import jax.numpy as jnp
from jax.experimental import pallas as pl


def kernel(input, table):
    raise NotImplementedError("write your pallas kernel here")


# speedup vs baseline: 1.8453x; 1.8453x over previous
import functools

import jax
import jax.numpy as jnp
from jax import lax
from jax.experimental import pallas as pl
from jax.experimental.pallas import tpu as pltpu
from jax.experimental.pallas import tpu_sc as plsc

EMBED = 64

_NC = 2
_NS = 16
_NW = _NC * _NS

_IDXW = 128
_CHUNK = 1024
_NJ = _CHUNK // _IDXW


def _make_gather(n_rows):
    assert n_rows % (_NW * _CHUNK) == 0
    bpw = n_rows // _NW
    nchunk = bpw // _CHUNK
    mesh = plsc.VectorSubcoreMesh(core_axis_name="c", subcore_axis_name="s")

    @functools.partial(
        pl.kernel,
        mesh=mesh,
        out_type=jax.ShapeDtypeStruct((n_rows, EMBED), jnp.float32),
        scratch_types=[
            pltpu.VMEM((_NJ, _IDXW), jnp.int32),
            pltpu.VMEM((_CHUNK, EMBED), jnp.float32),
            pltpu.SemaphoreType.DMA,
        ],
        compiler_params=pltpu.CompilerParams(use_tc_tiling_on_sc=False),
    )
    def k(idx_hbm, table_hbm, out_hbm, idx_v, rows_v, sem):
        wid = lax.axis_index("s") * _NC + lax.axis_index("c")
        idxrow0 = wid * (bpw // _IDXW)
        base = wid * bpw

        def chunk(c, carry):
            jr = pl.multiple_of(idxrow0 + c * _NJ, _NJ)
            pltpu.sync_copy(idx_hbm.at[pl.ds(jr, _NJ)], idx_v)
            handles = [
                pltpu.async_copy(
                    table_hbm.at[idx_v.at[j]],
                    rows_v.at[pl.ds(j * _IDXW, _IDXW)],
                    sem,
                )
                for j in range(_NJ)
            ]
            for h in handles:
                h.wait()
            start = pl.multiple_of(base + c * _CHUNK, _CHUNK)
            pltpu.sync_copy(rows_v, out_hbm.at[pl.ds(start, _CHUNK)])
            return carry

        lax.fori_loop(0, nchunk, chunk, 0)

    return k


def kernel(input, table):
    b, l = input.shape
    idx2 = input.reshape(-1, _IDXW).astype(jnp.int32)
    out = _make_gather(b * l)(idx2, table)
    return out.reshape(b, l, EMBED)


# speedup vs baseline: 1.8764x; 1.0169x over previous
import functools

import jax
import jax.numpy as jnp
from jax import lax
from jax.experimental import pallas as pl
from jax.experimental.pallas import tpu as pltpu
from jax.experimental.pallas import tpu_sc as plsc

EMBED = 64

_NC = 2
_NS = 16
_NW = _NC * _NS

_IDXW = 128
_CHUNK = 512
_NJ = _CHUNK // _IDXW


def _make_gather(n_rows):
    assert n_rows % (_NW * 2 * _CHUNK) == 0
    bpw = n_rows // _NW
    nchunk = bpw // _CHUNK
    ntrips = nchunk // 2
    nidxrow = bpw // _IDXW
    mesh = plsc.VectorSubcoreMesh(core_axis_name="c", subcore_axis_name="s")

    @functools.partial(
        pl.kernel,
        mesh=mesh,
        out_type=jax.ShapeDtypeStruct((n_rows, EMBED), jnp.float32),
        scratch_types=[
            pltpu.VMEM((nidxrow, _IDXW), jnp.int32),
            pltpu.VMEM((_CHUNK, EMBED), jnp.float32),
            pltpu.VMEM((_CHUNK, EMBED), jnp.float32),
            pltpu.SemaphoreType.DMA,
            pltpu.SemaphoreType.DMA,
        ],
        compiler_params=pltpu.CompilerParams(use_tc_tiling_on_sc=False),
    )
    def k(idx_hbm, table_hbm, out_hbm, idx_all, rows0, rows1, sem0, sem1):
        wid = lax.axis_index("s") * _NC + lax.axis_index("c")
        idxrow0 = wid * nidxrow
        base = wid * bpw

        pltpu.sync_copy(idx_hbm.at[pl.ds(idxrow0, nidxrow)], idx_all)

        def issue(c, rows, sem):
            for j in range(_NJ):
                pltpu.async_copy(
                    table_hbm.at[idx_all.at[c * _NJ + j]],
                    rows.at[pl.ds(j * _IDXW, _IDXW)],
                    sem,
                )

        def drain(c, rows, sem):
            for j in range(_NJ):
                pltpu.make_async_copy(
                    table_hbm.at[idx_all.at[c * _NJ + j]],
                    rows.at[pl.ds(j * _IDXW, _IDXW)],
                    sem,
                ).wait()

        def out_copy(c, rows):
            start = pl.multiple_of(base + c * _CHUNK, _CHUNK)
            pltpu.sync_copy(rows, out_hbm.at[pl.ds(start, _CHUNK)])

        issue(0, rows0, sem0)

        def body(t, carry):
            c0 = 2 * t
            issue(c0 + 1, rows1, sem1)
            drain(c0, rows0, sem0)
            out_copy(c0, rows0)
            issue(c0 + 2, rows0, sem0)
            drain(c0 + 1, rows1, sem1)
            out_copy(c0 + 1, rows1)
            return carry

        lax.fori_loop(0, ntrips - 1, body, 0)

        c0 = nchunk - 2
        issue(c0 + 1, rows1, sem1)
        drain(c0, rows0, sem0)
        out_copy(c0, rows0)
        drain(c0 + 1, rows1, sem1)
        out_copy(c0 + 1, rows1)

    return k


def kernel(input, table):
    b, l = input.shape
    idx2 = input.reshape(-1, _IDXW).astype(jnp.int32)
    out = _make_gather(b * l)(idx2, table)
    return out.reshape(b, l, EMBED)
